# baseline (speedup 1.0000x reference)
import jax
import jax.numpy as jnp
from jax import lax
from jax.experimental import pallas as pl
from jax.experimental.pallas import tpu as pltpu
from jax.experimental.pallas import tpu_sc as plsc

N = 10000
D = 128
DH = D // 2
E = 320000
NC = 2
NS = 16
ET = E // NS
KD = 80
CD = ET // KD
NB = 5
EHT = E // (NC * NS)
RT = N // NS
ZR = 125
HR = 640
NH = N // 16
RB = 1000


def _vector_mesh():
    return plsc.VectorSubcoreMesh(core_axis_name="c", subcore_axis_name="s",
                                  num_cores=NC, num_subcores=NS)


_SC_PARAMS = pltpu.CompilerParams(needs_layout_passes=False,
                                  use_tc_tiling_on_sc=False)


def _hist_call(eidx):
    @pl.kernel(
        out_type=jax.ShapeDtypeStruct((NC, NH, 16), jnp.float32),
        mesh=_vector_mesh(),
        compiler_params=_SC_PARAMS,
        scratch_types=[
            pltpu.VMEM((EHT,), jnp.int32),
            pltpu.VMEM((HR, 16), jnp.float32),
            pltpu.VMEM((1, HR), jnp.int32),
            pltpu.VMEM_SHARED((HR, 16), jnp.float32),
        ],
    )
    def hist_kernel(eidx_hbm, out_hbm, idx_v, hist_v, iota_v, acc_sh):
        c = lax.axis_index("c")
        s = lax.axis_index("s")
        w = c * NS + s
        pltpu.sync_copy(eidx_hbm.at[1, pl.ds(w * EHT, EHT)], idx_v)

        z16 = jnp.zeros((16,), jnp.float32)

        @pl.loop(0, HR)
        def _(i):
            hist_v[i, :] = z16

        @pl.loop(0, HR, step=16)
        def _(i):
            iota_v[0, pl.ds(i, 16)] = lax.iota(jnp.int32, 16) + i

        @pl.when(s == 0)
        def _():
            pltpu.sync_copy(hist_v, acc_sh)
        plsc.subcore_barrier()

        one16 = jnp.ones((16,), jnp.float32)

        @pl.loop(0, EHT // 16)
        def _(i):
            idx = idx_v[pl.ds(i * 16, 16)]
            plsc.addupdate_scatter(
                hist_v, [lax.shift_right_logical(idx, 4),
                         lax.bitwise_and(idx, 15)], one16)

        pltpu.sync_copy(hist_v, acc_sh.at[iota_v.at[0]], add=True)
        plsc.subcore_barrier()

        @pl.when(s == 0)
        def _():
            pltpu.sync_copy(acc_sh.at[pl.ds(0, NH)], out_hbm.at[c])

    return hist_kernel(eidx)


def _agg_call(ysp, eidx):
    @pl.kernel(
        out_type=jax.ShapeDtypeStruct((NC, N, DH), jnp.float32),
        mesh=_vector_mesh(),
        compiler_params=_SC_PARAMS,
        scratch_types=[
            pltpu.VMEM((ET,), jnp.int32),
            pltpu.VMEM((ET,), jnp.int32),
            pltpu.VMEM((NB * KD, DH), jnp.float32),
            pltpu.VMEM((ZR, DH), jnp.float32),
            pltpu.VMEM_SHARED((N, DH), jnp.float32),
            [pltpu.SemaphoreType.DMA] * NB,
            [pltpu.SemaphoreType.DMA] * NB,
            pltpu.SemaphoreType.DMA,
        ],
    )
    def agg_kernel(ys_hbm, eidx_hbm, out_hbm, srcv, dstv, rowbuf,
                   zbuf, acc_sh, gsem, ssem, msem):
        c = lax.axis_index("c")
        s = lax.axis_index("s")
        pltpu.sync_copy(eidx_hbm.at[0, pl.ds(s * ET, ET)], srcv)
        pltpu.sync_copy(eidx_hbm.at[1, pl.ds(s * ET, ET)], dstv)

        z16 = jnp.zeros((16,), jnp.float32)

        @pl.loop(0, ZR)
        def _(i):
            @pl.loop(0, DH, step=16)
            def _(j):
                zbuf[i, pl.ds(j, 16)] = z16

        zd = [pltpu.async_copy(zbuf, acc_sh.at[pl.ds(s * RT + ZR * k, ZR)],
                               msem) for k in range(RT // ZR)]
        for d in zd:
            d.wait()
        plsc.subcore_barrier()

        def _scatter_desc(b, cj):
            return pltpu.make_async_copy(
                rowbuf.at[pl.ds(b * KD, KD)],
                acc_sh.at[dstv.at[pl.ds(cj * KD, KD)]], ssem[b])

        @pl.loop(0, CD, step=NB)
        def _(j):
            gd = []
            for b in range(NB):
                @pl.when(j >= NB)
                def _():
                    _scatter_desc(b, j + b).wait()
                gd.append(pltpu.async_copy(
                    ys_hbm.at[c].at[srcv.at[pl.ds((j + b) * KD, KD)]],
                    rowbuf.at[pl.ds(b * KD, KD)], gsem[b]))
            for b in range(NB):
                gd[b].wait()
                pltpu.async_copy(
                    rowbuf.at[pl.ds(b * KD, KD)],
                    acc_sh.at[dstv.at[pl.ds((j + b) * KD, KD)]], ssem[b],
                    add=True)
        for b in range(NB):
            _scatter_desc(b, CD - NB + b).wait()

        plsc.subcore_barrier()
        pltpu.async_copy(acc_sh.at[pl.ds(s * RT, RT)],
                         out_hbm.at[c, pl.ds(s * RT, RT)], msem).wait()

    return agg_kernel(ysp, eidx)


def _dinv_col(hist_ref):
    deg = hist_ref[0] + hist_ref[1] + 1.0
    return lax.rsqrt(deg)


def _ln_mm_body(x_ref, w_ref, g_ref, bt_ref, hist_ref, ys_ref):
    xb = x_ref[...]
    mean = jnp.mean(xb, axis=1, keepdims=True)
    xc = xb - mean
    var = jnp.mean(xc * xc, axis=1, keepdims=True)
    xn = xc * lax.rsqrt(var + 1e-5) * g_ref[...] + bt_ref[...]
    xw = lax.dot_general(xn, w_ref[...], (((1,), (0,)), ((), ())),
                         precision=lax.Precision.HIGHEST,
                         preferred_element_type=jnp.float32)
    ys = xw * _dinv_col(hist_ref)
    ys_ref[0] = ys[:, :DH]
    ys_ref[1] = ys[:, DH:]


def _ln_mm_call(x, W, gamma, beta, hist2):
    return pl.pallas_call(
        _ln_mm_body,
        grid=(N // RB,),
        in_specs=[
            pl.BlockSpec((RB, D), lambda i: (i, 0)),
            pl.BlockSpec((D, D), lambda i: (0, 0)),
            pl.BlockSpec((1, D), lambda i: (0, 0)),
            pl.BlockSpec((1, D), lambda i: (0, 0)),
            pl.BlockSpec((NC, RB, 1), lambda i: (0, i, 0)),
        ],
        out_specs=pl.BlockSpec((NC, RB, DH), lambda i: (0, i, 0)),
        out_shape=jax.ShapeDtypeStruct((NC, N, DH), jnp.float32),
    )(x, W, gamma, beta, hist2)


def _fin_body(agg_ref, ys_ref, hist_ref, x_ref, b_ref, o_ref):
    t = jnp.concatenate([agg_ref[0] + ys_ref[0], agg_ref[1] + ys_ref[1]],
                        axis=1) * _dinv_col(hist_ref)
    o_ref[...] = jnp.maximum(t + b_ref[...] + x_ref[...], 0.0)


def _fin_call(agg, ysp, hist2, x, b2):
    return pl.pallas_call(
        _fin_body,
        grid=(N // RB,),
        in_specs=[
            pl.BlockSpec((NC, RB, DH), lambda i: (0, i, 0)),
            pl.BlockSpec((NC, RB, DH), lambda i: (0, i, 0)),
            pl.BlockSpec((NC, RB, 1), lambda i: (0, i, 0)),
            pl.BlockSpec((RB, D), lambda i: (i, 0)),
            pl.BlockSpec((1, D), lambda i: (0, 0)),
        ],
        out_specs=pl.BlockSpec((RB, D), lambda i: (i, 0)),
        out_shape=jax.ShapeDtypeStruct((N, D), jnp.float32),
    )(agg, ysp, hist2, x, b2)


def kernel(x, edge_index, edge_attr, h, batch, W, b, ln_gamma, ln_beta):
    eidx = edge_index.astype(jnp.int32)

    hist = _hist_call(eidx)
    hist2 = hist.reshape(NC, N, 1)
    ysp = _ln_mm_call(x, W, ln_gamma.reshape(1, D), ln_beta.reshape(1, D),
                      hist2)
    agg = _agg_call(ysp, eidx)
    out = _fin_call(agg, ysp, hist2, x, b.reshape(1, D))
    return out, h


# speedup vs baseline: 1.0472x; 1.0472x over previous
import jax
import jax.numpy as jnp
from jax import lax
from jax.experimental import pallas as pl
from jax.experimental.pallas import tpu as pltpu
from jax.experimental.pallas import tpu_sc as plsc

N = 10000
D = 128
DH = D // 2
E = 320000
NC = 2
NS = 16
ET = E // NS
KD = 80
CD = ET // KD
NB = 5
EHT = E // (NC * NS)
RT = N // NS
ZR = 125
HR = 640
NH = N // 16
RB = 2000


def _vector_mesh():
    return plsc.VectorSubcoreMesh(core_axis_name="c", subcore_axis_name="s",
                                  num_cores=NC, num_subcores=NS)


_SC_PARAMS = pltpu.CompilerParams(needs_layout_passes=False,
                                  use_tc_tiling_on_sc=False)


def _hist_call(eidx):
    @pl.kernel(
        out_type=jax.ShapeDtypeStruct((NH, 16), jnp.float32),
        mesh=_vector_mesh(),
        compiler_params=_SC_PARAMS,
        scratch_types=[
            pltpu.VMEM((ET,), jnp.int32),
            pltpu.VMEM((HR, 16), jnp.float32),
            pltpu.VMEM((1, HR), jnp.int32),
            pltpu.VMEM_SHARED((HR, 16), jnp.float32),
        ],
    )
    def hist_kernel(eidx_hbm, out_hbm, idx_v, hist_v, iota_v, acc_sh):
        c = lax.axis_index("c")
        s = lax.axis_index("s")
        pltpu.sync_copy(eidx_hbm.at[1, pl.ds(s * ET, ET)], idx_v)

        z16 = jnp.zeros((16,), jnp.float32)

        @pl.loop(0, HR)
        def _(i):
            hist_v[i, :] = z16

        @pl.loop(0, HR, step=16)
        def _(i):
            iota_v[0, pl.ds(i, 16)] = lax.iota(jnp.int32, 16) + i

        @pl.when(s == 0)
        def _():
            pltpu.sync_copy(hist_v, acc_sh)
        plsc.subcore_barrier()

        one16 = jnp.ones((16,), jnp.float32)

        @pl.when(c == 0)
        def _():
            @pl.loop(0, ET // 16)
            def _(i):
                idx = idx_v[pl.ds(i * 16, 16)]
                plsc.addupdate_scatter(
                    hist_v, [lax.shift_right_logical(idx, 4),
                             lax.bitwise_and(idx, 15)], one16)

            pltpu.sync_copy(hist_v, acc_sh.at[iota_v.at[0]], add=True)
        plsc.subcore_barrier()

        @pl.when((s == 0) & (c == 0))
        def _():
            pltpu.sync_copy(acc_sh.at[pl.ds(0, NH)], out_hbm)

    return hist_kernel(eidx)


def _agg_call(ysp, eidx):
    @pl.kernel(
        out_type=jax.ShapeDtypeStruct((NC, N, DH), jnp.float32),
        mesh=_vector_mesh(),
        compiler_params=_SC_PARAMS,
        scratch_types=[
            pltpu.VMEM((ET,), jnp.int32),
            pltpu.VMEM((ET,), jnp.int32),
            pltpu.VMEM((NB * KD, DH), jnp.float32),
            pltpu.VMEM((ZR, DH), jnp.float32),
            pltpu.VMEM_SHARED((N, DH), jnp.float32),
            [pltpu.SemaphoreType.DMA] * NB,
            [pltpu.SemaphoreType.DMA] * NB,
            pltpu.SemaphoreType.DMA,
        ],
    )
    def agg_kernel(ys_hbm, eidx_hbm, out_hbm, srcv, dstv, rowbuf,
                   zbuf, acc_sh, gsem, ssem, msem):
        c = lax.axis_index("c")
        s = lax.axis_index("s")
        pltpu.sync_copy(eidx_hbm.at[0, pl.ds(s * ET, ET)], srcv)
        pltpu.sync_copy(eidx_hbm.at[1, pl.ds(s * ET, ET)], dstv)

        z16 = jnp.zeros((16,), jnp.float32)

        @pl.loop(0, ZR)
        def _(i):
            @pl.loop(0, DH, step=16)
            def _(j):
                zbuf[i, pl.ds(j, 16)] = z16

        zd = [pltpu.async_copy(zbuf, acc_sh.at[pl.ds(s * RT + ZR * k, ZR)],
                               msem) for k in range(RT // ZR)]
        for d in zd:
            d.wait()
        plsc.subcore_barrier()

        def _scatter_desc(b, cj):
            return pltpu.make_async_copy(
                rowbuf.at[pl.ds(b * KD, KD)],
                acc_sh.at[dstv.at[pl.ds(cj * KD, KD)]], ssem[b])

        @pl.loop(0, CD, step=NB)
        def _(j):
            gd = []
            for b in range(NB):
                @pl.when(j >= NB)
                def _():
                    _scatter_desc(b, j + b).wait()
                gd.append(pltpu.async_copy(
                    ys_hbm.at[c].at[srcv.at[pl.ds((j + b) * KD, KD)]],
                    rowbuf.at[pl.ds(b * KD, KD)], gsem[b]))
            for b in range(NB):
                gd[b].wait()
                pltpu.async_copy(
                    rowbuf.at[pl.ds(b * KD, KD)],
                    acc_sh.at[dstv.at[pl.ds((j + b) * KD, KD)]], ssem[b],
                    add=True)
        for b in range(NB):
            _scatter_desc(b, CD - NB + b).wait()

        plsc.subcore_barrier()
        pltpu.async_copy(acc_sh.at[pl.ds(s * RT, RT)],
                         out_hbm.at[c, pl.ds(s * RT, RT)], msem).wait()

    return agg_kernel(ysp, eidx)


def _dinv_col(hist_ref):
    deg = hist_ref[...] + 1.0
    return lax.rsqrt(deg)


def _ln_mm_body(x_ref, w_ref, g_ref, bt_ref, hist_ref, ys_ref):
    xb = x_ref[...]
    mean = jnp.mean(xb, axis=1, keepdims=True)
    xc = xb - mean
    var = jnp.mean(xc * xc, axis=1, keepdims=True)
    xn = xc * lax.rsqrt(var + 1e-5) * g_ref[...] + bt_ref[...]
    xw = lax.dot_general(xn, w_ref[...], (((1,), (0,)), ((), ())),
                         precision=lax.Precision.HIGHEST,
                         preferred_element_type=jnp.float32)
    ys = xw * _dinv_col(hist_ref)
    ys_ref[0] = ys[:, :DH]
    ys_ref[1] = ys[:, DH:]


def _ln_mm_call(x, W, gamma, beta, hist2):
    return pl.pallas_call(
        _ln_mm_body,
        grid=(N // RB,),
        in_specs=[
            pl.BlockSpec((RB, D), lambda i: (i, 0)),
            pl.BlockSpec((D, D), lambda i: (0, 0)),
            pl.BlockSpec((1, D), lambda i: (0, 0)),
            pl.BlockSpec((1, D), lambda i: (0, 0)),
            pl.BlockSpec((RB, 1), lambda i: (i, 0)),
        ],
        out_specs=pl.BlockSpec((NC, RB, DH), lambda i: (0, i, 0)),
        out_shape=jax.ShapeDtypeStruct((NC, N, DH), jnp.float32),
    )(x, W, gamma, beta, hist2)


def _fin_body(agg_ref, ys_ref, hist_ref, x_ref, b_ref, o_ref):
    t = jnp.concatenate([agg_ref[0] + ys_ref[0], agg_ref[1] + ys_ref[1]],
                        axis=1) * _dinv_col(hist_ref)
    o_ref[...] = jnp.maximum(t + b_ref[...] + x_ref[...], 0.0)


def _fin_call(agg, ysp, hist2, x, b2):
    return pl.pallas_call(
        _fin_body,
        grid=(N // RB,),
        in_specs=[
            pl.BlockSpec((NC, RB, DH), lambda i: (0, i, 0)),
            pl.BlockSpec((NC, RB, DH), lambda i: (0, i, 0)),
            pl.BlockSpec((RB, 1), lambda i: (i, 0)),
            pl.BlockSpec((RB, D), lambda i: (i, 0)),
            pl.BlockSpec((1, D), lambda i: (0, 0)),
        ],
        out_specs=pl.BlockSpec((RB, D), lambda i: (i, 0)),
        out_shape=jax.ShapeDtypeStruct((N, D), jnp.float32),
    )(agg, ysp, hist2, x, b2)


def kernel(x, edge_index, edge_attr, h, batch, W, b, ln_gamma, ln_beta):
    eidx = edge_index.astype(jnp.int32)

    hist = _hist_call(eidx)
    hist2 = hist.reshape(N, 1)
    ysp = _ln_mm_call(x, W, ln_gamma.reshape(1, D), ln_beta.reshape(1, D),
                      hist2)
    agg = _agg_call(ysp, eidx)
    out = _fin_call(agg, ysp, hist2, x, b.reshape(1, D))
    return out, h
